# baseline (speedup 1.0000x reference)
import dataclasses
import functools

import jax
import jax.numpy as jnp
from jax import lax
from jax.experimental import pallas as pl
from jax.experimental.pallas import tpu as pltpu
from jax.experimental.pallas import tpu_sc as plsc

N_NODES = 10000
N_EDGES = 320000
N_GRAPHS = 64
N_CLASSES = 10

_NC = 2
_NS = 16
_NW = _NC * _NS
_K = 128
_CHUNKS_PER_W = 80
_HALF = _CHUNKS_PER_W // 2
_N_CHUNKS = _CHUNKS_PER_W * _NW
_E_PAD = _N_CHUNKS * _K

_ROWS_PER_TILE = 624
_ROWS_REM = N_NODES - _NS * _ROWS_PER_TILE

_HIGH = lax.Precision.HIGHEST


def _dot(a, b):
    return lax.dot_general(a, b, (((1,), (0,)), ((), ())),
                           preferred_element_type=jnp.float32,
                           precision=_HIGH)


def _mm(x, w):
    def body(x_ref, w_ref, o_ref):
        o_ref[...] = _dot(x_ref[...], w_ref[...])
    return pl.pallas_call(
        body,
        out_shape=jax.ShapeDtypeStruct((x.shape[0], w.shape[1]), jnp.float32),
    )(x, w)


def _fuse(acc, b, w):
    def body(a_ref, b_ref, w_ref, o_ref):
        h = jnp.maximum(a_ref[0] + a_ref[1] + b_ref[...], 0.0)
        o_ref[...] = _dot(h, w_ref[...])
    return pl.pallas_call(
        body,
        out_shape=jax.ShapeDtypeStruct((acc.shape[1], w.shape[1]), jnp.float32),
    )(acc, b.reshape(1, -1), w)


def _final(acc, b, batch2d, wlin, blin):
    def body(a_ref, b_ref, bt_ref, wl_ref, bl_ref, o_ref):
        out3 = a_ref[0] + a_ref[1] + b_ref[...]
        gi = lax.broadcasted_iota(jnp.int32, (N_NODES, N_GRAPHS), 1)
        onehot = (bt_ref[...] == gi).astype(jnp.float32)
        sums = lax.dot_general(onehot, out3, (((0,), (0,)), ((), ())),
                               preferred_element_type=jnp.float32,
                               precision=_HIGH)
        ones = jnp.ones((N_NODES, 1), jnp.float32)
        counts = lax.dot_general(onehot, ones, (((0,), (0,)), ((), ())),
                                 preferred_element_type=jnp.float32,
                                 precision=_HIGH)
        pooled = sums / jnp.maximum(counts, 1.0)
        o_ref[...] = _dot(pooled, wl_ref[...]) + bl_ref[...]
    return pl.pallas_call(
        body,
        out_shape=jax.ShapeDtypeStruct((N_GRAPHS, N_CLASSES), jnp.float32),
    )(acc, b.reshape(1, -1), batch2d, wlin, blin.reshape(1, -1))


def _make_scatter(d):
    mesh = plsc.VectorSubcoreMesh(core_axis_name="c", subcore_axis_name="s")
    cp = pltpu.CompilerParams()
    if "needs_layout_passes" in pltpu.CompilerParams.__dataclass_fields__:
        cp = dataclasses.replace(cp, needs_layout_passes=False)
    if d < 128 and "use_tc_tiling_on_sc" in pltpu.CompilerParams.__dataclass_fields__:
        cp = dataclasses.replace(cp, use_tc_tiling_on_sc=False)

    nj = d // 16

    @functools.partial(
        pl.kernel,
        compiler_params=cp,
        out_type=jax.ShapeDtypeStruct((_NC, N_NODES, d), jnp.float32),
        mesh=mesh,
        scratch_types=(
            [pltpu.VMEM((_K, d), jnp.float32) for _ in range(2)]
            + [pltpu.VMEM((_HALF, _K), jnp.int32),
               pltpu.VMEM((_HALF, _K), jnp.int32),
               pltpu.VMEM((_HALF, _K), jnp.float32)]
            + [pltpu.VMEM_SHARED((N_NODES, d), jnp.float32)]
            + [pltpu.SemaphoreType.DMA for _ in range(4)]
        ),
    )
    def sc_kernel(src_hbm, dst_hbm, w_hbm, h_hbm, z_hbm, out_hbm, *scr):
        rows = scr[0:2]
        sb, db, wb = scr[2], scr[3], scr[4]
        acc = scr[5]
        gsem = scr[6:8]
        ssem = scr[8:10]

        c = lax.axis_index("c")
        s = lax.axis_index("s")
        wid = s * _NC + c
        r0 = s * _ROWS_PER_TILE
        ch0 = wid * _CHUNKS_PER_W

        def start_gather(j, b):
            pltpu.async_copy(h_hbm.at[sb.at[j]], rows[b], gsem[b])

        def wait_gather(j, b):
            pltpu.make_async_copy(h_hbm.at[sb.at[j]], rows[b], gsem[b]).wait()

        def start_scatter(j, b):
            pltpu.async_copy(rows[b], acc.at[db.at[j]], ssem[b], add=True)

        def wait_scatter(j, b):
            pltpu.make_async_copy(rows[b], acc.at[db.at[j]], ssem[b]).wait()

        def multiply(j, b):
            @pl.loop(0, _K, step=4)
            def _(k0):
                for kk in range(4):
                    k = k0 + kk
                    wv = plsc.load_gather(
                        wb, [jnp.full((16,), 0, jnp.int32) + j,
                             jnp.full((16,), 0, jnp.int32) + k])
                    for jj in range(nj):
                        sl = (k, pl.ds(jj * 16, 16))
                        rows[b][sl] = rows[b][sl] * wv

        pltpu.sync_copy(z_hbm.at[pl.ds(r0, _ROWS_PER_TILE)],
                        acc.at[pl.ds(r0, _ROWS_PER_TILE)])

        @pl.when(s == _NS - 1)
        def _():
            pltpu.sync_copy(z_hbm.at[pl.ds(_NS * _ROWS_PER_TILE, _ROWS_REM)],
                            acc.at[pl.ds(_NS * _ROWS_PER_TILE, _ROWS_REM)])

        plsc.subcore_barrier()

        for half in range(2):
            hc0 = ch0 + half * _HALF
            pltpu.sync_copy(src_hbm.at[pl.ds(hc0, _HALF)], sb)
            pltpu.sync_copy(dst_hbm.at[pl.ds(hc0, _HALF)], db)
            pltpu.sync_copy(w_hbm.at[pl.ds(hc0, _HALF)], wb)

            start_gather(0, 0)

            @pl.loop(0, _HALF, step=2)
            def _(j0):
                for b in range(2):
                    j = j0 + b
                    wait_gather(j, b)
                    multiply(j, b)
                    start_scatter(j, b)

                    if b == 0:
                        @pl.when(j0 > 0)
                        def _():
                            wait_scatter(j - 1, 1)
                        start_gather(j + 1, 1)
                    else:
                        @pl.when(j0 < _HALF - 2)
                        def _():
                            wait_scatter(j - 1, 0)
                            start_gather(j + 1, 0)

            wait_scatter(_HALF - 2, 0)
            wait_scatter(_HALF - 1, 1)

        plsc.subcore_barrier()
        pltpu.sync_copy(acc.at[pl.ds(r0, _ROWS_PER_TILE)],
                        out_hbm.at[c, pl.ds(r0, _ROWS_PER_TILE)])

        @pl.when(s == _NS - 1)
        def _():
            pltpu.sync_copy(acc.at[pl.ds(_NS * _ROWS_PER_TILE, _ROWS_REM)],
                            out_hbm.at[c, pl.ds(_NS * _ROWS_PER_TILE, _ROWS_REM)])

    return sc_kernel


_scatter128 = _make_scatter(128)
_scatter64 = _make_scatter(64)


@jax.jit
def kernel(x, edge_index, batch, edge_weights, W1, b1, W2, b2, W3, b3,
           Wlin, blin):
    src = edge_index[0].astype(jnp.int32)
    dst = edge_index[1].astype(jnp.int32)
    pad = _E_PAD - N_EDGES
    src_p = jnp.concatenate([src, jnp.zeros((pad,), jnp.int32)]
                            ).reshape(_N_CHUNKS, _K)
    dst_p = jnp.concatenate([dst, jnp.zeros((pad,), jnp.int32)]
                            ).reshape(_N_CHUNKS, _K)
    w_p = jnp.concatenate([edge_weights.astype(jnp.float32),
                           jnp.zeros((pad,), jnp.float32)]
                          ).reshape(_N_CHUNKS, _K)

    z128 = jnp.zeros((N_NODES, 128), jnp.float32)
    z64 = jnp.zeros((N_NODES, 64), jnp.float32)
    batch2d = batch.astype(jnp.int32).reshape(N_NODES, 1)

    h1 = _mm(x, W1)
    a1 = _scatter128(src_p, dst_p, w_p, h1, z128)
    h2 = _fuse(a1, b1, W2)
    a2 = _scatter128(src_p, dst_p, w_p, h2, z128)
    h3 = _fuse(a2, b2, W3)
    a3 = _scatter64(src_p, dst_p, w_p, h3, z64)
    return _final(a3, b3, batch2d, Wlin, blin)


# speedup vs baseline: 1.2422x; 1.2422x over previous
import dataclasses
import functools

import jax
import jax.numpy as jnp
from jax import lax
from jax.experimental import pallas as pl
from jax.experimental.pallas import tpu as pltpu
from jax.experimental.pallas import tpu_sc as plsc

N_NODES = 10000
N_EDGES = 320000
N_GRAPHS = 64
N_CLASSES = 10

_NC = 2
_NS = 16
_NW = _NC * _NS
_K = 64
_CHUNKS_PER_W = 160
_HALF = _CHUNKS_PER_W // 4
_N_CHUNKS = _CHUNKS_PER_W * _NW
_E_PAD = _N_CHUNKS * _K

_ROWS_PER_TILE = 624
_ROWS_REM = N_NODES - _NS * _ROWS_PER_TILE

_HIGH = lax.Precision.HIGHEST


def _dot(a, b):
    return lax.dot_general(a, b, (((1,), (0,)), ((), ())),
                           preferred_element_type=jnp.float32,
                           precision=_HIGH)


def _mm(x, w):
    def body(x_ref, w_ref, o_ref):
        o_ref[...] = _dot(x_ref[...], w_ref[...])
    return pl.pallas_call(
        body,
        out_shape=jax.ShapeDtypeStruct((x.shape[0], w.shape[1]), jnp.float32),
    )(x, w)


def _fuse(acc, b, w):
    def body(a_ref, b_ref, w_ref, o_ref):
        h = jnp.maximum(a_ref[0] + a_ref[1] + b_ref[...], 0.0)
        o_ref[...] = _dot(h, w_ref[...])
    return pl.pallas_call(
        body,
        out_shape=jax.ShapeDtypeStruct((acc.shape[1], w.shape[1]), jnp.float32),
    )(acc, b.reshape(1, -1), w)


def _final(acc, b, batch2d, wlin, blin):
    def body(a_ref, b_ref, bt_ref, wl_ref, bl_ref, o_ref):
        out3 = a_ref[0] + a_ref[1] + b_ref[...]
        gi = lax.broadcasted_iota(jnp.int32, (N_NODES, N_GRAPHS), 1)
        onehot = (bt_ref[...] == gi).astype(jnp.float32)
        sums = lax.dot_general(onehot, out3, (((0,), (0,)), ((), ())),
                               preferred_element_type=jnp.float32,
                               precision=_HIGH)
        ones = jnp.ones((N_NODES, 1), jnp.float32)
        counts = lax.dot_general(onehot, ones, (((0,), (0,)), ((), ())),
                                 preferred_element_type=jnp.float32,
                                 precision=_HIGH)
        pooled = sums / jnp.maximum(counts, 1.0)
        o_ref[...] = _dot(pooled, wl_ref[...]) + bl_ref[...]
    return pl.pallas_call(
        body,
        out_shape=jax.ShapeDtypeStruct((N_GRAPHS, N_CLASSES), jnp.float32),
    )(acc, b.reshape(1, -1), batch2d, wlin, blin.reshape(1, -1))


def _make_scatter(d):
    mesh = plsc.VectorSubcoreMesh(core_axis_name="c", subcore_axis_name="s")
    cp = pltpu.CompilerParams()
    if "needs_layout_passes" in pltpu.CompilerParams.__dataclass_fields__:
        cp = dataclasses.replace(cp, needs_layout_passes=False)
    if d < 128 and "use_tc_tiling_on_sc" in pltpu.CompilerParams.__dataclass_fields__:
        cp = dataclasses.replace(cp, use_tc_tiling_on_sc=False)

    nj = d // 16

    @functools.partial(
        pl.kernel,
        compiler_params=cp,
        out_type=jax.ShapeDtypeStruct((_NC, N_NODES, d), jnp.float32),
        mesh=mesh,
        scratch_types=(
            [pltpu.VMEM((_K, d), jnp.float32) for _ in range(4)]
            + [pltpu.VMEM((_HALF, _K), jnp.int32),
               pltpu.VMEM((_HALF, _K), jnp.int32),
               pltpu.VMEM((_HALF, _K), jnp.float32)]
            + [pltpu.VMEM_SHARED((N_NODES, d), jnp.float32)]
            + [pltpu.SemaphoreType.DMA for _ in range(8)]
        ),
    )
    def sc_kernel(src_hbm, dst_hbm, w_hbm, h_hbm, z_hbm, out_hbm, *scr):
        rows = scr[0:4]
        sb, db, wb = scr[4], scr[5], scr[6]
        acc = scr[7]
        gsem = scr[8:12]
        ssem = scr[12:16]

        c = lax.axis_index("c")
        s = lax.axis_index("s")
        wid = s * _NC + c
        r0 = s * _ROWS_PER_TILE
        ch0 = wid * _CHUNKS_PER_W

        def start_gather(j, b):
            pltpu.async_copy(h_hbm.at[sb.at[j]], rows[b], gsem[b])

        def wait_gather(j, b):
            pltpu.make_async_copy(h_hbm.at[sb.at[j]], rows[b], gsem[b]).wait()

        def start_scatter(j, b):
            pltpu.async_copy(rows[b], acc.at[db.at[j]], ssem[b], add=True)

        def wait_scatter(j, b):
            pltpu.make_async_copy(rows[b], acc.at[db.at[j]], ssem[b]).wait()

        def multiply(j, b):
            @pl.loop(0, _K, step=4)
            def _(k0):
                for kk in range(4):
                    k = k0 + kk
                    wv = plsc.load_gather(
                        wb, [jnp.full((16,), 0, jnp.int32) + j,
                             jnp.full((16,), 0, jnp.int32) + k])
                    for jj in range(nj):
                        sl = (k, pl.ds(jj * 16, 16))
                        rows[b][sl] = rows[b][sl] * wv

        pltpu.sync_copy(z_hbm.at[pl.ds(r0, _ROWS_PER_TILE)],
                        acc.at[pl.ds(r0, _ROWS_PER_TILE)])

        @pl.when(s == _NS - 1)
        def _():
            pltpu.sync_copy(z_hbm.at[pl.ds(_NS * _ROWS_PER_TILE, _ROWS_REM)],
                            acc.at[pl.ds(_NS * _ROWS_PER_TILE, _ROWS_REM)])

        plsc.subcore_barrier()

        for half in range(4):
            hc0 = ch0 + half * _HALF
            pltpu.sync_copy(src_hbm.at[pl.ds(hc0, _HALF)], sb)
            pltpu.sync_copy(dst_hbm.at[pl.ds(hc0, _HALF)], db)
            pltpu.sync_copy(w_hbm.at[pl.ds(hc0, _HALF)], wb)

            start_gather(0, 0)
            start_gather(1, 1)

            @pl.loop(0, _HALF, step=4)
            def _(j0):
                for b in range(4):
                    j = j0 + b
                    wait_gather(j, b)
                    multiply(j, b)
                    start_scatter(j, b)

                    @pl.when(j >= 2)
                    def _():
                        wait_scatter(j - 2, (b + 2) % 4)

                    @pl.when(j + 2 < _HALF)
                    def _():
                        start_gather(j + 2, (b + 2) % 4)

            wait_scatter(_HALF - 2, 2)
            wait_scatter(_HALF - 1, 3)

        plsc.subcore_barrier()
        pltpu.sync_copy(acc.at[pl.ds(r0, _ROWS_PER_TILE)],
                        out_hbm.at[c, pl.ds(r0, _ROWS_PER_TILE)])

        @pl.when(s == _NS - 1)
        def _():
            pltpu.sync_copy(acc.at[pl.ds(_NS * _ROWS_PER_TILE, _ROWS_REM)],
                            out_hbm.at[c, pl.ds(_NS * _ROWS_PER_TILE, _ROWS_REM)])

    return sc_kernel


_scatter128 = _make_scatter(128)
_scatter64 = _make_scatter(64)


@jax.jit
def kernel(x, edge_index, batch, edge_weights, W1, b1, W2, b2, W3, b3,
           Wlin, blin):
    src = edge_index[0].astype(jnp.int32)
    dst = edge_index[1].astype(jnp.int32)
    pad = _E_PAD - N_EDGES
    src_p = jnp.concatenate([src, jnp.zeros((pad,), jnp.int32)]
                            ).reshape(_N_CHUNKS, _K)
    dst_p = jnp.concatenate([dst, jnp.zeros((pad,), jnp.int32)]
                            ).reshape(_N_CHUNKS, _K)
    w_p = jnp.concatenate([edge_weights.astype(jnp.float32),
                           jnp.zeros((pad,), jnp.float32)]
                          ).reshape(_N_CHUNKS, _K)

    z128 = jnp.zeros((N_NODES, 128), jnp.float32)
    z64 = jnp.zeros((N_NODES, 64), jnp.float32)
    batch2d = batch.astype(jnp.int32).reshape(N_NODES, 1)

    h1 = _mm(x, W1)
    a1 = _scatter128(src_p, dst_p, w_p, h1, z128)
    h2 = _fuse(a1, b1, W2)
    a2 = _scatter128(src_p, dst_p, w_p, h2, z128)
    h3 = _fuse(a2, b2, W3)
    a3 = _scatter64(src_p, dst_p, w_p, h3, z64)
    return _final(a3, b3, batch2d, Wlin, blin)


# speedup vs baseline: 1.3900x; 1.1190x over previous
import dataclasses
import functools

import jax
import jax.numpy as jnp
from jax import lax
from jax.experimental import pallas as pl
from jax.experimental.pallas import tpu as pltpu
from jax.experimental.pallas import tpu_sc as plsc

N_NODES = 10000
N_EDGES = 320000
N_GRAPHS = 64
N_CLASSES = 10

_NC = 2
_NS = 16
_NW = _NC * _NS
_K = 64
_CHUNKS_PER_W = 160
_HALF = _CHUNKS_PER_W // 4
_N_CHUNKS = _CHUNKS_PER_W * _NW
_E_PAD = _N_CHUNKS * _K

_ROWS_PER_TILE = 624
_ROWS_REM = N_NODES - _NS * _ROWS_PER_TILE

_HIGH = lax.Precision.HIGHEST


def _dot(a, b):
    return lax.dot_general(a, b, (((1,), (0,)), ((), ())),
                           preferred_element_type=jnp.float32,
                           precision=_HIGH)


def _mm(x, w):
    def body(x_ref, w_ref, o_ref):
        o_ref[...] = _dot(x_ref[...], w_ref[...])
    return pl.pallas_call(
        body,
        out_shape=jax.ShapeDtypeStruct((x.shape[0], w.shape[1]), jnp.float32),
    )(x, w)


def _fuse(acc, b, w):
    def body(a_ref, b_ref, w_ref, o_ref):
        h = jnp.maximum(a_ref[0] + a_ref[1] + b_ref[...], 0.0)
        o_ref[...] = _dot(h, w_ref[...])
    return pl.pallas_call(
        body,
        out_shape=jax.ShapeDtypeStruct((acc.shape[1], w.shape[1]), jnp.float32),
    )(acc, b.reshape(1, -1), w)


def _final(acc, b, batch2d, wlin, blin):
    def body(a_ref, b_ref, bt_ref, wl_ref, bl_ref, o_ref):
        out3 = a_ref[0] + a_ref[1] + b_ref[...]
        gi = lax.broadcasted_iota(jnp.int32, (N_NODES, N_GRAPHS), 1)
        onehot = (bt_ref[...] == gi).astype(jnp.float32)
        sums = lax.dot_general(onehot, out3, (((0,), (0,)), ((), ())),
                               preferred_element_type=jnp.float32,
                               precision=_HIGH)
        ones = jnp.ones((N_NODES, 1), jnp.float32)
        counts = lax.dot_general(onehot, ones, (((0,), (0,)), ((), ())),
                                 preferred_element_type=jnp.float32,
                                 precision=_HIGH)
        pooled = sums / jnp.maximum(counts, 1.0)
        o_ref[...] = _dot(pooled, wl_ref[...]) + bl_ref[...]
    return pl.pallas_call(
        body,
        out_shape=jax.ShapeDtypeStruct((N_GRAPHS, N_CLASSES), jnp.float32),
    )(acc, b.reshape(1, -1), batch2d, wlin, blin.reshape(1, -1))


def _make_scatter(d):
    mesh = plsc.VectorSubcoreMesh(core_axis_name="c", subcore_axis_name="s")
    cp = pltpu.CompilerParams()
    if "needs_layout_passes" in pltpu.CompilerParams.__dataclass_fields__:
        cp = dataclasses.replace(cp, needs_layout_passes=False)
    if d < 128 and "use_tc_tiling_on_sc" in pltpu.CompilerParams.__dataclass_fields__:
        cp = dataclasses.replace(cp, use_tc_tiling_on_sc=False)

    nj = d // 16

    @functools.partial(
        pl.kernel,
        compiler_params=cp,
        out_type=jax.ShapeDtypeStruct((_NC, N_NODES, d), jnp.float32),
        mesh=mesh,
        scratch_types=(
            [pltpu.VMEM((_K, d), jnp.float32) for _ in range(4)]
            + [pltpu.VMEM((_HALF, _K), jnp.int32),
               pltpu.VMEM((_HALF, _K), jnp.float32)]
            + [pltpu.VMEM((_K,), jnp.int32) for _ in range(4)]
            + [pltpu.VMEM_SHARED((N_NODES, d), jnp.float32)]
            + [pltpu.SemaphoreType.DMA for _ in range(12)]
        ),
    )
    def sc_kernel(src_hbm, dst_hbm, w_hbm, h_hbm, z_hbm, out_hbm, *scr):
        rows = scr[0:4]
        sb, wb = scr[4], scr[5]
        dv = scr[6:10]
        acc = scr[10]
        gsem = scr[11:15]
        ssem = scr[15:19]
        dsem = scr[19:23]

        c = lax.axis_index("c")
        s = lax.axis_index("s")
        wid = s * _NC + c
        r0 = s * _ROWS_PER_TILE
        ch0 = wid * _CHUNKS_PER_W

        def start_gather(j, b):
            pltpu.async_copy(h_hbm.at[sb.at[j]], rows[b], gsem[b])

        def wait_gather(j, b):
            pltpu.make_async_copy(h_hbm.at[sb.at[j]], rows[b], gsem[b]).wait()

        def start_dst(g, b):
            pltpu.async_copy(dst_hbm.at[pl.ds(g * _K, _K)], dv[b], dsem[b])

        def wait_dst(g, b):
            pltpu.make_async_copy(dst_hbm.at[pl.ds(g * _K, _K)],
                                  dv[b], dsem[b]).wait()

        def start_scatter(b):
            pltpu.async_copy(rows[b], acc.at[dv[b]], ssem[b], add=True)

        def wait_scatter(b):
            pltpu.make_async_copy(rows[b], acc.at[dv[b]], ssem[b]).wait()

        def multiply(j, b):
            @pl.loop(0, _K, step=4)
            def _(k0):
                for kk in range(4):
                    k = k0 + kk
                    wv = plsc.load_gather(
                        wb, [jnp.full((16,), 0, jnp.int32) + j,
                             jnp.full((16,), 0, jnp.int32) + k])
                    for jj in range(nj):
                        sl = (k, pl.ds(jj * 16, 16))
                        rows[b][sl] = rows[b][sl] * wv

        pltpu.sync_copy(z_hbm.at[pl.ds(r0, _ROWS_PER_TILE)],
                        acc.at[pl.ds(r0, _ROWS_PER_TILE)])

        @pl.when(s == _NS - 1)
        def _():
            pltpu.sync_copy(z_hbm.at[pl.ds(_NS * _ROWS_PER_TILE, _ROWS_REM)],
                            acc.at[pl.ds(_NS * _ROWS_PER_TILE, _ROWS_REM)])

        plsc.subcore_barrier()

        for half in range(4):
            hc0 = ch0 + half * _HALF
            pltpu.sync_copy(src_hbm.at[pl.ds(hc0, _HALF)], sb)
            pltpu.sync_copy(w_hbm.at[pl.ds(hc0, _HALF)], wb)

            start_gather(0, 0)
            start_gather(1, 1)
            start_dst(hc0, 0)
            start_dst(hc0 + 1, 1)

            @pl.loop(0, _HALF, step=4)
            def _(j0):
                for b in range(4):
                    j = j0 + b
                    wait_gather(j, b)
                    multiply(j, b)
                    wait_dst(hc0 + j, b)
                    start_scatter(b)

                    @pl.when(j >= 2)
                    def _():
                        wait_scatter((b + 2) % 4)

                    @pl.when(j + 2 < _HALF)
                    def _():
                        start_gather(j + 2, (b + 2) % 4)
                        start_dst(hc0 + j + 2, (b + 2) % 4)

            wait_scatter(2)
            wait_scatter(3)

        plsc.subcore_barrier()
        pltpu.sync_copy(acc.at[pl.ds(r0, _ROWS_PER_TILE)],
                        out_hbm.at[c, pl.ds(r0, _ROWS_PER_TILE)])

        @pl.when(s == _NS - 1)
        def _():
            pltpu.sync_copy(acc.at[pl.ds(_NS * _ROWS_PER_TILE, _ROWS_REM)],
                            out_hbm.at[c, pl.ds(_NS * _ROWS_PER_TILE, _ROWS_REM)])

    return sc_kernel


_scatter128 = _make_scatter(128)
_scatter64 = _make_scatter(64)


@jax.jit
def kernel(x, edge_index, batch, edge_weights, W1, b1, W2, b2, W3, b3,
           Wlin, blin):
    src = edge_index[0].astype(jnp.int32)
    dst = edge_index[1].astype(jnp.int32)
    pad = _E_PAD - N_EDGES
    src_p = jnp.concatenate([src, jnp.zeros((pad,), jnp.int32)]
                            ).reshape(_N_CHUNKS, _K)
    dst_p = jnp.concatenate([dst, jnp.zeros((pad,), jnp.int32)])
    w_p = jnp.concatenate([edge_weights.astype(jnp.float32),
                           jnp.zeros((pad,), jnp.float32)]
                          ).reshape(_N_CHUNKS, _K)

    z128 = jnp.zeros((N_NODES, 128), jnp.float32)
    z64 = jnp.zeros((N_NODES, 64), jnp.float32)
    batch2d = batch.astype(jnp.int32).reshape(N_NODES, 1)

    h1 = _mm(x, W1)
    a1 = _scatter128(src_p, dst_p, w_p, h1, z128)
    h2 = _fuse(a1, b1, W2)
    a2 = _scatter128(src_p, dst_p, w_p, h2, z128)
    h3 = _fuse(a2, b2, W3)
    a3 = _scatter64(src_p, dst_p, w_p, h3, z64)
    return _final(a3, b3, batch2d, Wlin, blin)
